# Initial kernel scaffold; baseline (speedup 1.0000x reference)
import jax
import jax.numpy as jnp
from jax.experimental import pallas as pl


def kernel(x, x_coords, mask):
    raise NotImplementedError("write your pallas kernel here")


# speedup vs baseline: 1.8369x; 1.8369x over previous
import functools

import jax
import jax.numpy as jnp
from jax import lax
from jax.experimental import pallas as pl
from jax.experimental.pallas import tpu as pltpu
from jax.experimental.pallas import tpu_sc as plsc

_B, _N, _F = 8, 8192, 512
_R, _C = 64, 128
_DEPTHS = 10
_ALIGNS = 9

def _build_consts():
    if True:
        rkey = jax.random.key(42)
        old = jnp.ones((_B, 3, 1), jnp.float32)
        abcs = []
        for i in range(_DEPTHS):
            abc = jax.random.normal(jax.random.fold_in(rkey, 2 * i),
                                    (_B, 3, 1), jnp.float32)
            na = abc / jnp.maximum(
                jnp.linalg.norm(abc, axis=1, keepdims=True), 1e-12)
            no = old / jnp.maximum(
                jnp.linalg.norm(old, axis=1, keepdims=True), 1e-12)
            ang = jnp.sum(na * no, axis=1, keepdims=True)
            s = jnp.sign(ang)
            s = jnp.where(s == 0, 1.0, s)
            abc = abc * s
            old = abc
            abcs.append(abc[:, :, 0])
        abc_all = jnp.stack(abcs, axis=1)
        us = []
        for i in range(_ALIGNS):
            k_align = jax.random.fold_in(rkey, 2 * i + 1)
            kl, kr = jax.random.split(k_align)
            kl1, kl2 = jax.random.split(kl)
            kr1, kr2 = jax.random.split(kr)
            for kk in (kl1, kl2, kr1, kr2):
                us.append(jax.random.uniform(kk, (_B, _N)))
        unif = jnp.stack(us, axis=1).reshape(_B, 4 * _ALIGNS, _R, _C)
        return abc_all, unif


def _pick_point(u, msk, p0, p1, p2, idx):
    s = u + (1.0 - msk) * 1000000000.0
    mval = jnp.min(s)
    cand = jnp.where(s == mval, idx, jnp.int32(2 ** 30))
    jsel = jnp.min(cand)
    oh = (idx == jsel).astype(jnp.float32)
    return jnp.sum(p0 * oh), jnp.sum(p1 * oh), jnp.sum(p2 * oh)


def _score_kernel(abc_ref, coords_ref, unif_ref, mask_ref, key_ref):
    m = mask_ref[0]
    p0 = coords_ref[0, 0]
    p1 = coords_ref[0, 1]
    p2 = coords_ref[0, 2]
    row = lax.broadcasted_iota(jnp.int32, (_R, _C), 0)
    col = lax.broadcasted_iota(jnp.int32, (_R, _C), 1)
    idx = row * _C + col
    total = jnp.zeros((_R, _C), jnp.float32)
    nm = jnp.sum(m)
    kk = lax.broadcasted_iota(jnp.int32, (_C, 32), 0)
    cc = lax.broadcasted_iota(jnp.int32, (_C, 32), 1)
    ktgt = 16 * (cc // 4) + 3 * (cc % 4)
    dn = (((1,), (0,)), ((), ()))
    spreads = []
    for j in range(4):
        ci = lax.broadcasted_iota(jnp.int32, (_C, _C), 0)
        ki = lax.broadcasted_iota(jnp.int32, (_C, _C), 1)
        inj = (ci >= 32 * j) & (ci < 32 * j + 32)
        kt = 16 * ((ci - 32 * j) // 4) + 3 * ((ci - 32 * j) % 4)
        spreads.append([
            (inj & (ki == kt + i)).astype(jnp.float32) for i in range(3)])
    for i in range(_DEPTHS):
        a = abc_ref[0, i, 0]
        bb = abc_ref[0, i, 1]
        c = abc_ref[0, i, 2]
        bmat = (jnp.where(kk == ktgt, a, 0.0)
                + jnp.where(kk == ktgt + 1, bb, 0.0)
                + jnp.where(kk == ktgt + 2, c, 0.0))
        outs = []
        for j in range(4):
            amat = (lax.dot_general(p0, spreads[j][0], dn,
                                    preferred_element_type=jnp.float32)
                    + lax.dot_general(p1, spreads[j][1], dn,
                                      preferred_element_type=jnp.float32)
                    + lax.dot_general(p2, spreads[j][2], dn,
                                      preferred_element_type=jnp.float32))
            outs.append(lax.dot_general(amat, bmat, dn,
                                        preferred_element_type=jnp.float32))
        proj = jnp.concatenate(outs, axis=1)
        d = -(jnp.sum(proj * m) / (nm + 1e-09))
        score = jnp.sign(proj + d)
        total = total + score * float(512 >> i)
        if i < _ALIGNS:
            mask_l = jnp.minimum(jnp.maximum(score * -1.0, 0.0), m)
            mask_r = jnp.minimum(jnp.maximum(score, 0.0), m)
            l10, l11, l12 = _pick_point(unif_ref[0, 4 * i + 0], mask_l,
                                        p0, p1, p2, idx)
            l20, l21, l22 = _pick_point(unif_ref[0, 4 * i + 1], mask_l,
                                        p0, p1, p2, idx)
            r10, r11, r12 = _pick_point(unif_ref[0, 4 * i + 2], mask_r,
                                        p0, p1, p2, idx)
            r20, r21, r22 = _pick_point(unif_ref[0, 4 * i + 3], mask_r,
                                        p0, p1, p2, idx)
            dl0, dl1, dl2 = ((l10 + l20) / 2.0, (l11 + l21) / 2.0,
                             (l12 + l22) / 2.0)
            dr0, dr1, dr2 = ((r10 + r20) / 2.0, (r11 + r21) / 2.0,
                             (r12 + r22) / 2.0)
            p0 = (p0 - dl0) * mask_l + (p0 - dr0) * mask_r
            p1 = (p1 - dl1) * mask_l + (p1 - dr1) * mask_r
            p2 = (p2 - dl2) * mask_l + (p2 - dr2) * mask_r
    total = total + 2048.0 * (1.0 - m)
    key_ref[0] = total.astype(jnp.int32) * _N + idx


def _rank_kernel(key_ref, keycol_ref, rank_ref, rankg_ref):
    b = pl.program_id(0)
    kc = keycol_ref[0]
    ones = jnp.ones((_C, 1), jnp.float32)

    def body(r, cnt):
        qrow = key_ref[0, pl.ds(r, 1), :]
        lt = (qrow < kc).astype(jnp.float32)
        c = lax.dot_general(lt, ones, (((1,), (0,)), ((), ())),
                            preferred_element_type=jnp.float32)
        return cnt + c

    cnt = lax.fori_loop(0, _R, body, jnp.zeros((_N, 1), jnp.float32))
    rank = cnt.astype(jnp.int32)
    rank_ref[0] = rank
    rankg_ref[0] = rank + b * _N


def _rank_call(abc, coords_t, unif, mask2d):
    key2d = pl.pallas_call(
        _score_kernel,
        grid=(_B,),
        in_specs=[
            pl.BlockSpec((1, _DEPTHS, 3), lambda b: (b, 0, 0),
                         memory_space=pltpu.SMEM),
            pl.BlockSpec((1, 3, _R, _C), lambda b: (b, 0, 0, 0)),
            pl.BlockSpec((1, 4 * _ALIGNS, _R, _C), lambda b: (b, 0, 0, 0)),
            pl.BlockSpec((1, _R, _C), lambda b: (b, 0, 0)),
        ],
        out_specs=pl.BlockSpec((1, _R, _C), lambda b: (b, 0, 0)),
        out_shape=jax.ShapeDtypeStruct((_B, _R, _C), jnp.int32),
    )(abc, coords_t, unif, mask2d)
    keycol = key2d.reshape(_B, _N, 1)
    return pl.pallas_call(
        _rank_kernel,
        grid=(_B,),
        in_specs=[
            pl.BlockSpec((1, _R, _C), lambda b: (b, 0, 0)),
            pl.BlockSpec((1, _N, 1), lambda b: (b, 0, 0)),
        ],
        out_specs=[
            pl.BlockSpec((1, _N, 1), lambda b: (b, 0, 0)),
            pl.BlockSpec((1, _N, 1), lambda b: (b, 0, 0)),
        ],
        out_shape=[
            jax.ShapeDtypeStruct((_B, _N, 1), jnp.int32),
            jax.ShapeDtypeStruct((_B, _N, 1), jnp.int32),
        ],
    )(key2d, keycol)


_ROWS = _B * _N
_NW = 32
_RPW = _ROWS // _NW
_CH = 128
_NCHUNK = _RPW // _CH


def _make_scatter():
    mesh = plsc.VectorSubcoreMesh(core_axis_name="c", subcore_axis_name="s")

    @functools.partial(
        pl.kernel,
        mesh=mesh,
        out_type=jax.ShapeDtypeStruct((_ROWS, _F), jnp.float32),
        scratch_types=[
            pltpu.VMEM((_CH,), jnp.int32),
            pltpu.VMEM((_CH, _F), jnp.float32),
            pltpu.SemaphoreType.DMA,
        ],
    )
    def scatter_k(x_hbm, idx_hbm, out_hbm, idx_v, rows_v, sem):
        wid = lax.axis_index("s") * 2 + lax.axis_index("c")
        base = wid * _RPW

        def body(ci, carry):
            off = base + ci * _CH
            pltpu.sync_copy(idx_hbm.at[pl.ds(off, _CH)], idx_v)
            pltpu.sync_copy(x_hbm.at[pl.ds(off, _CH)], rows_v)
            pltpu.async_copy(rows_v, out_hbm.at[idx_v], sem).wait()
            return carry

        lax.fori_loop(0, _NCHUNK, body, 0)

    return scatter_k


def kernel(x, x_coords, mask):
    abc, unif = _build_consts()
    coords_t = jnp.transpose(x_coords, (0, 2, 1)).reshape(_B, 3, _R, _C)
    mask2d = mask.reshape(_B, _R, _C)
    rank_col, rankg_col = _rank_call(abc, coords_t, unif, mask2d)
    rank = rank_col.reshape(_B, _N)
    x_out = _make_scatter()(x.reshape(_ROWS, _F), rankg_col.reshape(_ROWS))
    return x_out.reshape(_B, _N, _F), rank
